# baseline (speedup 1.0000x reference)
import jax
import jax.numpy as jnp
from jax import lax
from jax.experimental import pallas as pl

NCLASS = 1000
N = 16384
BLK = 512


def _onehot(x_ref, o_ref):
    xb = x_ref[...]
    rows = lax.broadcasted_iota(jnp.int32, (NCLASS, BLK), 0)
    o_ref[...] = jnp.where(rows == xb[None, :], 1.0, 0.0).astype(jnp.float32)


def kernel(x):
    x = x.astype(jnp.int32)
    z = pl.pallas_call(
        _onehot,
        out_shape=jax.ShapeDtypeStruct((NCLASS, N), jnp.float32),
        grid=(N // BLK,),
        in_specs=[pl.BlockSpec((BLK,), lambda i: (i,))],
        out_specs=pl.BlockSpec((NCLASS, BLK), lambda i: (0, i)),
    )(x)
    return z.T


# speedup vs baseline: 1.1922x; 1.1922x over previous
import jax
import jax.numpy as jnp
from jax import lax
from jax.experimental import pallas as pl

NCLASS = 1000
N = 16384
RBLK = 40


def _onehot(x_ref, o_ref):
    xb = x_ref[...]
    r0 = pl.program_id(0) * RBLK
    rows = lax.broadcasted_iota(jnp.int32, (RBLK, N), 0) + r0
    o_ref[...] = jnp.where(rows == xb[None, :], 1.0, 0.0).astype(jnp.float32)


def kernel(x):
    x = x.astype(jnp.int32)
    z = pl.pallas_call(
        _onehot,
        out_shape=jax.ShapeDtypeStruct((NCLASS, N), jnp.float32),
        grid=(NCLASS // RBLK,),
        in_specs=[pl.BlockSpec((N,), lambda i: (0,))],
        out_specs=pl.BlockSpec((RBLK, N), lambda i: (i, 0)),
    )(x)
    return z.T


# speedup vs baseline: 1.3003x; 1.0906x over previous
import jax
import jax.numpy as jnp
from jax import lax
from jax.experimental import pallas as pl

NCLASS = 1000
N = 16384
BLK = 1024


def _onehot(x_ref, o_ref):
    xb = x_ref[...]
    rows = lax.broadcasted_iota(jnp.int32, (NCLASS, BLK), 0)
    o_ref[...] = jnp.where(rows == xb[None, :], 1.0, 0.0).astype(jnp.float32)


def kernel(x):
    x = x.astype(jnp.int32)
    z = pl.pallas_call(
        _onehot,
        out_shape=jax.ShapeDtypeStruct((NCLASS, N), jnp.float32),
        grid=(N // BLK,),
        in_specs=[pl.BlockSpec((BLK,), lambda i: (i,))],
        out_specs=pl.BlockSpec((NCLASS, BLK), lambda i: (0, i)),
    )(x)
    return z.T
